# Initial kernel scaffold; baseline (speedup 1.0000x reference)
import jax
import jax.numpy as jnp
from jax.experimental import pallas as pl


def kernel(attn_bias, spatial_pos, attn_edge_type, spatial_pos_table, edge_table, virtual_dist):
    raise NotImplementedError("write your pallas kernel here")


# speedup vs baseline: 18.8817x; 18.8817x over previous
import functools

import jax
import jax.numpy as jnp
from jax import lax
from jax.experimental import pallas as pl
from jax.experimental.pallas import tpu as pltpu
from jax.experimental.pallas import tpu_sc as plsc

B, N, H = 32, 128, 32
NP1 = N + 1
PP = 136
PLANEP = NP1 * PP
NSP = 512
NET = 1537
NEF = 3
ROWS = 8
NSTRIP = N // ROWS
SROW = (ROWS + 1) * PP
NC, NS, L = 2, 16, 16

_mesh = plsc.VectorSubcoreMesh(core_axis_name="c", subcore_axis_name="s")


@functools.partial(
    pl.kernel,
    mesh=_mesh,
    compiler_params=pltpu.CompilerParams(use_tc_tiling_on_sc=False,
                                         needs_layout_passes=False),
    out_type=jax.ShapeDtypeStruct((B, H, PLANEP), jnp.float32),
    scratch_types=[
        pltpu.VMEM((H * NSP,), jnp.float32),
        pltpu.VMEM((H * NET,), jnp.float32),
        pltpu.VMEM((H,), jnp.float32),
        pltpu.VMEM((ROWS * N,), jnp.int32),
        pltpu.VMEM((ROWS * N * NEF,), jnp.int32),
        pltpu.VMEM(((ROWS + 1) * PP,), jnp.float32),
        pltpu.VMEM((H * SROW,), jnp.float32),
        pltpu.SemaphoreType.DMA,
    ],
)
def _sc_bias_kernel(bias_hbm, spidx_hbm, etidx_hbm, spT_hbm, etT_hbm, vd_hbm,
                    out_hbm, spT_v, etT_v, vd_v, spi_v, eti_v, bias_v,
                    strip_v, sem):
    w = lax.axis_index("s") * NC + lax.axis_index("c")
    b = w

    pltpu.sync_copy(spT_hbm, spT_v)
    pltpu.sync_copy(etT_hbm, etT_v)
    pltpu.sync_copy(vd_hbm, vd_v)

    iota = lax.iota(jnp.int32, L)
    iota3 = iota * 3
    lane_lt8 = iota < 8
    lane0 = iota == 0

    def task(k, _):
        i0 = ROWS * k

        pltpu.sync_copy(spidx_hbm.at[b, pl.ds(i0 * N, ROWS * N)], spi_v)
        pltpu.sync_copy(etidx_hbm.at[b, pl.ds(i0 * N * NEF, ROWS * N * NEF)],
                        eti_v)
        pltpu.sync_copy(bias_hbm.at[b, pl.ds(i0 * PP, (ROWS + 1) * PP)],
                        bias_v)

        def row(ri, _):
            lr = ri + 1
            for c in range(8):
                j0 = 16 * c
                spi = spi_v[pl.ds(ri * N + j0, L)]
                ebase = iota3 + (ri * (N * NEF) + j0 * NEF)
                e0 = plsc.load_gather(eti_v, [ebase])
                e1 = plsc.load_gather(eti_v, [ebase + 1])
                e2 = plsc.load_gather(eti_v, [ebase + 2])
                bv = bias_v[pl.ds(lr * PP + 1 + j0, L)]

                def hgrp(hq, _):
                    for u in range(4):
                        h = 4 * hq + u
                        sp = plsc.load_gather(spT_v, [spi + h * NSP])
                        f0 = plsc.load_gather(etT_v, [e0 + h * NET])
                        f1 = plsc.load_gather(etT_v, [e1 + h * NET])
                        f2 = plsc.load_gather(etT_v, [e2 + h * NET])
                        strip_v[pl.ds(h * SROW + lr * PP + 1 + j0, L)] = (
                            bv + sp + f0 + f1 + f2)
                    return 0

                lax.fori_loop(0, 8, hgrp, 0, unroll=False)
            return 0

        lax.fori_loop(0, ROWS, row, 0, unroll=False)

        def border(h, _):
            th = plsc.load_gather(vd_v, [jnp.full((L,), h, jnp.int32)])
            ridx = (iota + 1) * PP
            c0 = plsc.load_gather(bias_v, [ridx], mask=lane_lt8)
            plsc.store_scatter(strip_v, [ridx + h * SROW], c0 + th,
                               mask=lane_lt8)

            @pl.when(k == 0)
            def _():
                for c in range(8):
                    j0 = 16 * c
                    strip_v[pl.ds(h * SROW + j0, L)] = (
                        bias_v[pl.ds(j0, L)] + th)
                lastc = jnp.full((L,), N, jnp.int32)
                r0 = plsc.load_gather(bias_v, [lastc], mask=lane0)
                plsc.store_scatter(strip_v, [lastc + h * SROW], r0 + th,
                                   mask=lane0)
            return 0

        lax.fori_loop(0, H, border, 0, unroll=False)

        def fire(h, _):
            @pl.when(k == 0)
            def _():
                pltpu.async_copy(strip_v.at[pl.ds(h * SROW, SROW)],
                                 out_hbm.at[b, h, pl.ds(0, SROW)], sem)

            @pl.when(k != 0)
            def _():
                pltpu.async_copy(
                    strip_v.at[pl.ds(h * SROW + PP, ROWS * PP)],
                    out_hbm.at[b, h, pl.ds((i0 + 1) * PP, ROWS * PP)], sem)
            return 0

        lax.fori_loop(0, H, fire, 0, unroll=False)

        def drain(h, _):
            @pl.when(k == 0)
            def _():
                pltpu.make_async_copy(strip_v.at[pl.ds(h * SROW, SROW)],
                                      out_hbm.at[b, h, pl.ds(0, SROW)],
                                      sem).wait()

            @pl.when(k != 0)
            def _():
                pltpu.make_async_copy(
                    strip_v.at[pl.ds(h * SROW + PP, ROWS * PP)],
                    out_hbm.at[b, h, pl.ds((i0 + 1) * PP, ROWS * PP)],
                    sem).wait()
            return 0

        lax.fori_loop(0, H, drain, 0, unroll=False)
        return 0

    lax.fori_loop(0, NSTRIP, task, 0, unroll=False)


def kernel(attn_bias, spatial_pos, attn_edge_type, spatial_pos_table,
           edge_table, virtual_dist):
    spT = spatial_pos_table.T.reshape(-1)
    etT = (edge_table * (1.0 / 3.0)).T.reshape(-1)
    vd = virtual_dist.reshape(-1).astype(jnp.float32)
    spi = spatial_pos.astype(jnp.int32).reshape(B, N * N)
    eti = attn_edge_type.astype(jnp.int32).reshape(B, N * N * NEF)
    bias = jnp.pad(attn_bias, ((0, 0), (0, 0), (0, PP - NP1))).reshape(
        B, PLANEP)
    out = _sc_bias_kernel(bias, spi, eti, spT, etT, vd)
    return out.reshape(B, H, NP1, PP)[:, :, :, :NP1]


# speedup vs baseline: 25.6107x; 1.3564x over previous
import functools

import jax
import jax.numpy as jnp
from jax import lax
from jax.experimental import pallas as pl
from jax.experimental.pallas import tpu as pltpu
from jax.experimental.pallas import tpu_sc as plsc

B, N, H = 32, 128, 32
NP1 = N + 1
PP = 136
PLANEP = NP1 * PP
NSP = 512
NET = 1537
ETP = 1544
NEF = 3
ROWS = 8
NSTRIP = N // ROWS
SROW = (ROWS + 1) * PP
NC, NS, L = 2, 16, 16

_mesh = plsc.VectorSubcoreMesh(core_axis_name="c", subcore_axis_name="s")


@functools.partial(
    pl.kernel,
    mesh=_mesh,
    compiler_params=pltpu.CompilerParams(use_tc_tiling_on_sc=False,
                                         needs_layout_passes=False),
    out_type=jax.ShapeDtypeStruct((B, H, PLANEP), jnp.float32),
    scratch_types=[
        pltpu.VMEM((H * NSP,), jnp.float32),
        pltpu.VMEM((H * ETP,), jnp.float32),
        pltpu.VMEM((H,), jnp.float32),
        pltpu.VMEM((ROWS * N,), jnp.int32),
        pltpu.VMEM((ROWS * N * NEF,), jnp.int32),
        pltpu.VMEM(((ROWS + 1) * PP,), jnp.float32),
        pltpu.VMEM((H * SROW,), jnp.float32),
        pltpu.SemaphoreType.DMA,
    ],
)
def _sc_bias_kernel(bias_hbm, spidx_hbm, etidx_hbm, spT_hbm, etT_hbm, vd_hbm,
                    out_hbm, spT_v, etT_v, vd_v, spi_v, eti_v, bias_v,
                    strip_v, sem):
    w = lax.axis_index("s") * NC + lax.axis_index("c")
    b = w

    pltpu.sync_copy(spT_hbm, spT_v)
    pltpu.sync_copy(etT_hbm, etT_v)
    pltpu.sync_copy(vd_hbm, vd_v)

    iota = lax.iota(jnp.int32, L)
    iota3 = iota * 3
    lane_lt8 = iota < 8
    lane0 = iota == 0

    def task(k, _):
        i0 = ROWS * k

        pltpu.sync_copy(spidx_hbm.at[b, pl.ds(i0 * N, ROWS * N)], spi_v)
        pltpu.sync_copy(etidx_hbm.at[b, pl.ds(i0 * N * NEF, ROWS * N * NEF)],
                        eti_v)
        pltpu.sync_copy(bias_hbm.at[b, pl.ds(i0 * PP, (ROWS + 1) * PP)],
                        bias_v)

        def row(ri, _):
            lr = ri + 1
            for c in range(8):
                j0 = 16 * c
                spi = spi_v[pl.ds(ri * N + j0, L)]
                ebase = iota3 + (ri * (N * NEF) + j0 * NEF)
                e0 = plsc.load_gather(eti_v, [ebase])
                e1 = plsc.load_gather(eti_v, [ebase + 1])
                e2 = plsc.load_gather(eti_v, [ebase + 2])
                bv = bias_v[pl.ds(lr * PP + 1 + j0, L)]

                @plsc.parallel_loop(0, H, unroll=4)
                def _(h):
                    sp = plsc.load_gather(
                        spT_v.at[pl.ds(h * NSP, NSP)], [spi])
                    f0 = plsc.load_gather(
                        etT_v.at[pl.ds(h * ETP, ETP)], [e0])
                    f1 = plsc.load_gather(
                        etT_v.at[pl.ds(h * ETP, ETP)], [e1])
                    f2 = plsc.load_gather(
                        etT_v.at[pl.ds(h * ETP, ETP)], [e2])
                    strip_v[pl.ds(h * SROW + lr * PP + 1 + j0, L)] = (
                        (bv + sp) + ((f0 + f1) + f2))
            return 0

        lax.fori_loop(0, ROWS, row, 0, unroll=False)

        def border(h, _):
            th = plsc.load_gather(vd_v, [jnp.full((L,), h, jnp.int32)])
            ridx = (iota + 1) * PP
            c0 = plsc.load_gather(bias_v, [ridx], mask=lane_lt8)
            plsc.store_scatter(strip_v, [ridx + h * SROW], c0 + th,
                               mask=lane_lt8)

            @pl.when(k == 0)
            def _():
                for c in range(8):
                    j0 = 16 * c
                    strip_v[pl.ds(h * SROW + j0, L)] = (
                        bias_v[pl.ds(j0, L)] + th)
                lastc = jnp.full((L,), N, jnp.int32)
                r0 = plsc.load_gather(bias_v, [lastc], mask=lane0)
                plsc.store_scatter(strip_v, [lastc + h * SROW], r0 + th,
                                   mask=lane0)
            return 0

        lax.fori_loop(0, H, border, 0, unroll=False)

        def fire(h, _):
            @pl.when(k == 0)
            def _():
                pltpu.async_copy(strip_v.at[pl.ds(h * SROW, SROW)],
                                 out_hbm.at[b, h, pl.ds(0, SROW)], sem)

            @pl.when(k != 0)
            def _():
                pltpu.async_copy(
                    strip_v.at[pl.ds(h * SROW + PP, ROWS * PP)],
                    out_hbm.at[b, h, pl.ds((i0 + 1) * PP, ROWS * PP)], sem)
            return 0

        lax.fori_loop(0, H, fire, 0, unroll=False)

        def drain(h, _):
            @pl.when(k == 0)
            def _():
                pltpu.make_async_copy(strip_v.at[pl.ds(h * SROW, SROW)],
                                      out_hbm.at[b, h, pl.ds(0, SROW)],
                                      sem).wait()

            @pl.when(k != 0)
            def _():
                pltpu.make_async_copy(
                    strip_v.at[pl.ds(h * SROW + PP, ROWS * PP)],
                    out_hbm.at[b, h, pl.ds((i0 + 1) * PP, ROWS * PP)],
                    sem).wait()
            return 0

        lax.fori_loop(0, H, drain, 0, unroll=False)
        return 0

    lax.fori_loop(0, NSTRIP, task, 0, unroll=False)


def kernel(attn_bias, spatial_pos, attn_edge_type, spatial_pos_table,
           edge_table, virtual_dist):
    spT = spatial_pos_table.T.reshape(-1)
    etT = jnp.pad((edge_table * (1.0 / 3.0)).T,
                  ((0, 0), (0, ETP - NET))).reshape(-1)
    vd = virtual_dist.reshape(-1).astype(jnp.float32)
    spi = spatial_pos.astype(jnp.int32).reshape(B, N * N)
    eti = attn_edge_type.astype(jnp.int32).reshape(B, N * N * NEF)
    bias = jnp.pad(attn_bias, ((0, 0), (0, 0), (0, PP - NP1))).reshape(
        B, PLANEP)
    out = _sc_bias_kernel(bias, spi, eti, spT, etT, vd)
    return out.reshape(B, H, NP1, PP)[:, :, :, :NP1]


# speedup vs baseline: 29.2661x; 1.1427x over previous
import functools

import jax
import jax.numpy as jnp
from jax import lax
from jax.experimental import pallas as pl
from jax.experimental.pallas import tpu as pltpu
from jax.experimental.pallas import tpu_sc as plsc

B, N, H = 32, 128, 32
NP1 = N + 1
NSP = 512
NET = 1537
ETP = 1544
NEF = 3
ROWS = 8
NSTRIP = N // ROWS
NC, NS, L = 2, 16, 16
HB = 8

_mesh = plsc.VectorSubcoreMesh(core_axis_name="c", subcore_axis_name="s")


@functools.partial(
    pl.kernel,
    mesh=_mesh,
    compiler_params=pltpu.CompilerParams(use_tc_tiling_on_sc=False,
                                         needs_layout_passes=False),
    out_type=jax.ShapeDtypeStruct((B, H, N, N), jnp.float32),
    scratch_types=[
        pltpu.VMEM((H * NSP,), jnp.float32),
        pltpu.VMEM((H * ETP,), jnp.float32),
        pltpu.VMEM((ROWS * N,), jnp.int32),
        pltpu.VMEM((ROWS * N * NEF,), jnp.int32),
        pltpu.VMEM((H * ROWS, N), jnp.float32),
        pltpu.SemaphoreType.DMA,
    ],
)
def _sc_tsum_kernel(spidx_hbm, etidx_hbm, spT_hbm, etT_hbm,
                    out_hbm, spT_v, etT_v, spi_v, eti_v, strip_v, sem):
    w = lax.axis_index("s") * NC + lax.axis_index("c")
    b = w

    pltpu.sync_copy(spT_hbm, spT_v)
    pltpu.sync_copy(etT_hbm, etT_v)

    iota = lax.iota(jnp.int32, L)
    iota3 = iota * 3

    def task(k, _):
        i0 = ROWS * k

        pltpu.sync_copy(spidx_hbm.at[b, pl.ds(i0 * N, ROWS * N)], spi_v)
        pltpu.sync_copy(etidx_hbm.at[b, pl.ds(i0 * N * NEF, ROWS * N * NEF)],
                        eti_v)

        def row(ri, _):
            for c in range(8):
                j0 = 16 * c
                spi = spi_v[pl.ds(ri * N + j0, L)]
                ebase = iota3 + (ri * (N * NEF) + j0 * NEF)
                e0 = plsc.load_gather(eti_v, [ebase])
                e1 = plsc.load_gather(eti_v, [ebase + 1])
                e2 = plsc.load_gather(eti_v, [ebase + 2])

                @plsc.parallel_loop(0, H, unroll=4)
                def _(h):
                    sp = plsc.load_gather(
                        spT_v.at[pl.ds(h * NSP, NSP)], [spi])
                    f0 = plsc.load_gather(
                        etT_v.at[pl.ds(h * ETP, ETP)], [e0])
                    f1 = plsc.load_gather(
                        etT_v.at[pl.ds(h * ETP, ETP)], [e1])
                    f2 = plsc.load_gather(
                        etT_v.at[pl.ds(h * ETP, ETP)], [e2])
                    strip_v[h * ROWS + ri, pl.ds(j0, L)] = (
                        (sp + f0) + (f1 + f2))
            return 0

        lax.fori_loop(0, ROWS, row, 0, unroll=False)

        def fire(h, _):
            pltpu.async_copy(strip_v.at[pl.ds(h * ROWS, ROWS), :],
                             out_hbm.at[b, h, pl.ds(i0, ROWS), :], sem)
            return 0

        lax.fori_loop(0, H, fire, 0, unroll=False)

        def drain(h, _):
            pltpu.make_async_copy(strip_v.at[pl.ds(h * ROWS, ROWS), :],
                                  out_hbm.at[b, h, pl.ds(i0, ROWS), :],
                                  sem).wait()
            return 0

        lax.fori_loop(0, H, drain, 0, unroll=False)
        return 0

    lax.fori_loop(0, NSTRIP, task, 0, unroll=False)


def _tc_assemble_body(bias_ref, border_ref, tsum_ref, o_ref):
    for u in range(HB):
        inner = jnp.pad(tsum_ref[0, u], ((1, 0), (1, 0)))
        o_ref[0, u] = bias_ref[0] + border_ref[u] + inner


_tc_assemble = pl.pallas_call(
    _tc_assemble_body,
    grid=(B, H // HB),
    in_specs=[
        pl.BlockSpec((1, NP1, NP1), lambda b, hg: (b, 0, 0)),
        pl.BlockSpec((HB, NP1, NP1), lambda b, hg: (hg, 0, 0)),
        pl.BlockSpec((1, HB, N, N), lambda b, hg: (b, hg, 0, 0)),
    ],
    out_specs=pl.BlockSpec((1, HB, NP1, NP1), lambda b, hg: (b, hg, 0, 0)),
    out_shape=jax.ShapeDtypeStruct((B, H, NP1, NP1), jnp.float32),
)


def kernel(attn_bias, spatial_pos, attn_edge_type, spatial_pos_table,
           edge_table, virtual_dist):
    spT = spatial_pos_table.T.reshape(-1)
    etT = jnp.pad((edge_table * (1.0 / 3.0)).T,
                  ((0, 0), (0, ETP - NET))).reshape(-1)
    spi = spatial_pos.astype(jnp.int32).reshape(B, N * N)
    eti = attn_edge_type.astype(jnp.int32).reshape(B, N * N * NEF)
    tsum = _sc_tsum_kernel(spi, eti, spT, etT)

    t = virtual_dist.reshape(H, 1, 1).astype(jnp.float32)
    rr = lax.broadcasted_iota(jnp.int32, (1, NP1, NP1), 1)
    cc = lax.broadcasted_iota(jnp.int32, (1, NP1, NP1), 2)
    border = jnp.where((rr == 0) | (cc == 0), t, 0.0)

    return _tc_assemble(attn_bias, border, tsum)


# speedup vs baseline: 31.2008x; 1.0661x over previous
import functools

import jax
import jax.numpy as jnp
from jax import lax
from jax.experimental import pallas as pl
from jax.experimental.pallas import tpu as pltpu
from jax.experimental.pallas import tpu_sc as plsc

B, N, H = 32, 128, 32
NP1 = N + 1
NSP = 512
NET = 1537
ETP = 1544
NEF = 3
ROWS = 8
NSTRIP = N // ROWS
NC, NS, L = 2, 16, 16
HB = 8

_mesh = plsc.VectorSubcoreMesh(core_axis_name="c", subcore_axis_name="s")


@functools.partial(
    pl.kernel,
    mesh=_mesh,
    compiler_params=pltpu.CompilerParams(use_tc_tiling_on_sc=False,
                                         needs_layout_passes=False),
    out_type=jax.ShapeDtypeStruct((B, H, N, N), jnp.float32),
    scratch_types=[
        pltpu.VMEM((H * NSP,), jnp.float32),
        pltpu.VMEM((H * ETP,), jnp.float32),
        pltpu.VMEM((ROWS * N,), jnp.int32),
        pltpu.VMEM((ROWS * N * NEF,), jnp.int32),
        pltpu.VMEM((H * ROWS, N), jnp.float32),
        pltpu.SemaphoreType.DMA,
    ],
)
def _sc_tsum_kernel(spidx_hbm, etidx_hbm, spT_hbm, etT_hbm,
                    out_hbm, spT_v, etT_v, spi_v, eti_v, strip_v, sem):
    w = lax.axis_index("s") * NC + lax.axis_index("c")
    b = w

    pltpu.sync_copy(spT_hbm, spT_v)
    pltpu.sync_copy(etT_hbm, etT_v)

    iota = lax.iota(jnp.int32, L)
    iota3 = iota * 3

    def task(k, _):
        i0 = ROWS * k

        pltpu.sync_copy(spidx_hbm.at[b, pl.ds(i0 * N, ROWS * N)], spi_v)
        pltpu.sync_copy(etidx_hbm.at[b, pl.ds(i0 * N * NEF, ROWS * N * NEF)],
                        eti_v)

        def row(ri, _):
            for c in range(8):
                j0 = 16 * c
                spi = spi_v[pl.ds(ri * N + j0, L)]
                ebase = iota3 + (ri * (N * NEF) + j0 * NEF)
                e0 = plsc.load_gather(eti_v, [ebase])
                e1 = plsc.load_gather(eti_v, [ebase + 1])
                e2 = plsc.load_gather(eti_v, [ebase + 2])

                @plsc.parallel_loop(0, H, unroll=4)
                def _(h):
                    sp = plsc.load_gather(
                        spT_v.at[pl.ds(h * NSP, NSP)], [spi])
                    f0 = plsc.load_gather(
                        etT_v.at[pl.ds(h * ETP, ETP)], [e0])
                    f1 = plsc.load_gather(
                        etT_v.at[pl.ds(h * ETP, ETP)], [e1])
                    f2 = plsc.load_gather(
                        etT_v.at[pl.ds(h * ETP, ETP)], [e2])
                    strip_v[h * ROWS + ri, pl.ds(j0, L)] = (
                        (sp + f0) + (f1 + f2))
            return 0

        lax.fori_loop(0, ROWS, row, 0, unroll=False)

        def fire(h, _):
            pltpu.async_copy(strip_v.at[pl.ds(h * ROWS, ROWS), :],
                             out_hbm.at[b, h, pl.ds(i0, ROWS), :], sem)
            return 0

        lax.fori_loop(0, H, fire, 0, unroll=False)

        def drain(h, _):
            pltpu.make_async_copy(strip_v.at[pl.ds(h * ROWS, ROWS), :],
                                  out_hbm.at[b, h, pl.ds(i0, ROWS), :],
                                  sem).wait()
            return 0

        lax.fori_loop(0, H, drain, 0, unroll=False)
        return 0

    lax.fori_loop(0, NSTRIP, task, 0, unroll=False)


def _tc_assemble_body(bias_ref, t_ref, tsum_ref, o_ref):
    rr = lax.broadcasted_iota(jnp.int32, (NP1, NP1), 0)
    cc = lax.broadcasted_iota(jnp.int32, (NP1, NP1), 1)
    mask = (rr == 0) | (cc == 0)
    hg = pl.program_id(1)
    for u in range(HB):
        t_u = t_ref[hg * HB + u]
        base = bias_ref[0] + jnp.where(mask, t_u, 0.0)
        o_ref[0, u] = base + jnp.pad(tsum_ref[0, u], ((1, 0), (1, 0)))


_tc_assemble = pl.pallas_call(
    _tc_assemble_body,
    grid=(B, H // HB),
    in_specs=[
        pl.BlockSpec((1, NP1, NP1), lambda b, hg: (b, 0, 0)),
        pl.BlockSpec(memory_space=pltpu.SMEM),
        pl.BlockSpec((1, HB, N, N), lambda b, hg: (b, hg, 0, 0)),
    ],
    out_specs=pl.BlockSpec((1, HB, NP1, NP1), lambda b, hg: (b, hg, 0, 0)),
    out_shape=jax.ShapeDtypeStruct((B, H, NP1, NP1), jnp.float32),
)


def kernel(attn_bias, spatial_pos, attn_edge_type, spatial_pos_table,
           edge_table, virtual_dist):
    spT = spatial_pos_table.T.reshape(-1)
    etT = jnp.pad((edge_table * (1.0 / 3.0)).T,
                  ((0, 0), (0, ETP - NET))).reshape(-1)
    spi = spatial_pos.astype(jnp.int32).reshape(B, N * N)
    eti = attn_edge_type.astype(jnp.int32).reshape(B, N * N * NEF)
    tsum = _sc_tsum_kernel(spi, eti, spT, etT)

    t = virtual_dist.reshape(H).astype(jnp.float32)
    return _tc_assemble(attn_bias, t, tsum)


# speedup vs baseline: 34.4630x; 1.1046x over previous
import functools

import jax
import jax.numpy as jnp
from jax import lax
from jax.experimental import pallas as pl
from jax.experimental.pallas import tpu as pltpu
from jax.experimental.pallas import tpu_sc as plsc

B, N, H = 32, 128, 32
NP1 = N + 1
NSP = 512
NET = 1537
ETP = 1544
NEF = 3
ROWS = 8
NSTRIP = N // ROWS
NC, NS, L = 2, 16, 16
HB = 8

_mesh = plsc.VectorSubcoreMesh(core_axis_name="c", subcore_axis_name="s")


@functools.partial(
    pl.kernel,
    mesh=_mesh,
    compiler_params=pltpu.CompilerParams(use_tc_tiling_on_sc=False,
                                         needs_layout_passes=False),
    out_type=jax.ShapeDtypeStruct((B, H, N, N), jnp.float32),
    scratch_types=[
        pltpu.VMEM((H // 2 * NSP,), jnp.int32),
        pltpu.VMEM((H // 2 * ETP,), jnp.int32),
        pltpu.VMEM((ROWS * N,), jnp.int32),
        pltpu.VMEM((ROWS * N * NEF,), jnp.int32),
        pltpu.VMEM((H * ROWS, N), jnp.float32),
        pltpu.SemaphoreType.DMA,
    ],
)
def _sc_tsum_kernel(spidx_hbm, etidx_hbm, spT_hbm, etT_hbm,
                    out_hbm, spT_v, etT_v, spi_v, eti_v, strip_v, sem):
    w = lax.axis_index("s") * NC + lax.axis_index("c")
    b = w

    pltpu.sync_copy(spT_hbm, spT_v)
    pltpu.sync_copy(etT_hbm, etT_v)

    iota = lax.iota(jnp.int32, L)
    iota3 = iota * 3

    def task(k, _):
        i0 = ROWS * k

        pltpu.sync_copy(spidx_hbm.at[b, pl.ds(i0 * N, ROWS * N)], spi_v)
        pltpu.sync_copy(etidx_hbm.at[b, pl.ds(i0 * N * NEF, ROWS * N * NEF)],
                        eti_v)

        def row(ri, _):
            for c in range(8):
                j0 = 16 * c
                spi = spi_v[pl.ds(ri * N + j0, L)]
                ebase = iota3 + (ri * (N * NEF) + j0 * NEF)
                e0 = plsc.load_gather(eti_v, [ebase])
                e1 = plsc.load_gather(eti_v, [ebase + 1])
                e2 = plsc.load_gather(eti_v, [ebase + 2])

                @plsc.parallel_loop(0, H // 2, unroll=4)
                def _(hp):
                    gs = plsc.load_gather(
                        spT_v.at[pl.ds(hp * NSP, NSP)], [spi])
                    g0 = plsc.load_gather(
                        etT_v.at[pl.ds(hp * ETP, ETP)], [e0])
                    g1 = plsc.load_gather(
                        etT_v.at[pl.ds(hp * ETP, ETP)], [e1])
                    g2 = plsc.load_gather(
                        etT_v.at[pl.ds(hp * ETP, ETP)], [e2])
                    sa, sb = plsc.unpack(plsc.bitcast(gs, jnp.bfloat16),
                                         format=plsc.PackFormat.INTERLEAVED)
                    a0, b0 = plsc.unpack(plsc.bitcast(g0, jnp.bfloat16),
                                         format=plsc.PackFormat.INTERLEAVED)
                    a1, b1 = plsc.unpack(plsc.bitcast(g1, jnp.bfloat16),
                                         format=plsc.PackFormat.INTERLEAVED)
                    a2, b2 = plsc.unpack(plsc.bitcast(g2, jnp.bfloat16),
                                         format=plsc.PackFormat.INTERLEAVED)
                    strip_v[(2 * hp) * ROWS + ri, pl.ds(j0, L)] = (
                        (sa + a0) + (a1 + a2))
                    strip_v[(2 * hp + 1) * ROWS + ri, pl.ds(j0, L)] = (
                        (sb + b0) + (b1 + b2))
            return 0

        lax.fori_loop(0, ROWS, row, 0, unroll=False)

        def fire(h, _):
            pltpu.async_copy(strip_v.at[pl.ds(h * ROWS, ROWS), :],
                             out_hbm.at[b, h, pl.ds(i0, ROWS), :], sem)
            return 0

        lax.fori_loop(0, H, fire, 0, unroll=False)

        def drain(h, _):
            pltpu.make_async_copy(strip_v.at[pl.ds(h * ROWS, ROWS), :],
                                  out_hbm.at[b, h, pl.ds(i0, ROWS), :],
                                  sem).wait()
            return 0

        lax.fori_loop(0, H, drain, 0, unroll=False)
        return 0

    lax.fori_loop(0, NSTRIP, task, 0, unroll=False)


def _tc_assemble_body(bias_ref, t_ref, tsum_ref, o_ref):
    rr = lax.broadcasted_iota(jnp.int32, (NP1, NP1), 0)
    cc = lax.broadcasted_iota(jnp.int32, (NP1, NP1), 1)
    mask = (rr == 0) | (cc == 0)
    hg = pl.program_id(1)
    for u in range(HB):
        t_u = t_ref[hg * HB + u]
        base = bias_ref[0] + jnp.where(mask, t_u, 0.0)
        o_ref[0, u] = base + jnp.pad(tsum_ref[0, u], ((1, 0), (1, 0)))


_tc_assemble = pl.pallas_call(
    _tc_assemble_body,
    grid=(B, H // HB),
    in_specs=[
        pl.BlockSpec((1, NP1, NP1), lambda b, hg: (b, 0, 0)),
        pl.BlockSpec(memory_space=pltpu.SMEM),
        pl.BlockSpec((1, HB, N, N), lambda b, hg: (b, hg, 0, 0)),
    ],
    out_specs=pl.BlockSpec((1, HB, NP1, NP1), lambda b, hg: (b, hg, 0, 0)),
    out_shape=jax.ShapeDtypeStruct((B, H, NP1, NP1), jnp.float32),
)


def _pack_pairs(T):
    tb = lax.bitcast_convert_type(T.astype(jnp.bfloat16),
                                  jnp.uint16).astype(jnp.uint32)
    lo = tb[0::2, :]
    hi = tb[1::2, :]
    return (lo | (hi << 16)).astype(jnp.int32).reshape(-1)


def kernel(attn_bias, spatial_pos, attn_edge_type, spatial_pos_table,
           edge_table, virtual_dist):
    spT = _pack_pairs(spatial_pos_table.T)
    etT = _pack_pairs(jnp.pad((edge_table * (1.0 / 3.0)).T,
                              ((0, 0), (0, ETP - NET))))
    spi = spatial_pos.astype(jnp.int32).reshape(B, N * N)
    eti = attn_edge_type.astype(jnp.int32).reshape(B, N * N * NEF)
    tsum = _sc_tsum_kernel(spi, eti, spT, etT)

    t = virtual_dist.reshape(H).astype(jnp.float32)
    return _tc_assemble(attn_bias, t, tsum)


# speedup vs baseline: 37.6893x; 1.0936x over previous
import functools

import jax
import jax.numpy as jnp
from jax import lax
from jax.experimental import pallas as pl
from jax.experimental.pallas import tpu as pltpu
from jax.experimental.pallas import tpu_sc as plsc

B, N, H = 32, 128, 32
NP1 = N + 1
NSP = 512
NET = 1537
ETP = 1544
NEF = 3
ROWS = 8
NSTRIP = N // ROWS
NC, NS, L = 2, 16, 16
HB = 8

_mesh = plsc.VectorSubcoreMesh(core_axis_name="c", subcore_axis_name="s")


@functools.partial(
    pl.kernel,
    mesh=_mesh,
    compiler_params=pltpu.CompilerParams(use_tc_tiling_on_sc=False,
                                         needs_layout_passes=False),
    out_type=jax.ShapeDtypeStruct((B, H, N, N), jnp.float32),
    scratch_types=[
        pltpu.VMEM((H // 2 * NSP,), jnp.int32),
        pltpu.VMEM((H // 2 * ETP,), jnp.int32),
        pltpu.VMEM((2, ROWS, N), jnp.int32),
        pltpu.VMEM((2, ROWS * N * NEF,), jnp.int32),
        pltpu.VMEM((2, H * ROWS, N), jnp.float32),
        pltpu.SemaphoreType.DMA,
        pltpu.SemaphoreType.DMA,
    ],
)
def _sc_tsum_kernel(spidx_hbm, etidx_hbm, spT_hbm, etT_hbm,
                    out_hbm, spT_v, etT_v, spi_v, eti_v, strip_v,
                    sem, sem_idx):
    w = lax.axis_index("s") * NC + lax.axis_index("c")
    b = w

    pltpu.sync_copy(spT_hbm, spT_v)
    pltpu.sync_copy(etT_hbm, etT_v)

    iota = lax.iota(jnp.int32, L)
    iota3 = iota * 3

    pltpu.sync_copy(spidx_hbm.at[b, pl.ds(0, ROWS), :], spi_v.at[0])
    pltpu.sync_copy(etidx_hbm.at[b, pl.ds(0, ROWS * N * NEF)], eti_v.at[0])

    def task(k, _):
        i0 = ROWS * k
        p = lax.rem(k, 2)
        q = lax.rem(k + 1, 2)

        @pl.when(k < NSTRIP - 1)
        def _():
            pltpu.async_copy(spidx_hbm.at[b, pl.ds(i0 + ROWS, ROWS), :],
                             spi_v.at[q], sem_idx)
            pltpu.async_copy(
                etidx_hbm.at[b, pl.ds((i0 + ROWS) * N * NEF, ROWS * N * NEF)],
                eti_v.at[q], sem_idx)

        @pl.when(k >= 2)
        def _():
            def dr(h, _):
                pltpu.make_async_copy(
                    strip_v.at[p, pl.ds(h * ROWS, ROWS), :],
                    out_hbm.at[b, h, pl.ds(0, ROWS), :], sem).wait()
                return 0
            lax.fori_loop(0, H, dr, 0, unroll=False)

        def row(ri, _):
            for c in range(8):
                j0 = 16 * c
                spi = spi_v[p, ri, pl.ds(j0, L)]
                ebase = iota3 + (ri * (N * NEF) + j0 * NEF)
                e0 = plsc.load_gather(eti_v.at[p], [ebase])
                e1 = plsc.load_gather(eti_v.at[p], [ebase + 1])
                e2 = plsc.load_gather(eti_v.at[p], [ebase + 2])

                @plsc.parallel_loop(0, H // 2, unroll=4)
                def _(hp):
                    gs = plsc.load_gather(
                        spT_v.at[pl.ds(hp * NSP, NSP)], [spi])
                    g0 = plsc.load_gather(
                        etT_v.at[pl.ds(hp * ETP, ETP)], [e0])
                    g1 = plsc.load_gather(
                        etT_v.at[pl.ds(hp * ETP, ETP)], [e1])
                    g2 = plsc.load_gather(
                        etT_v.at[pl.ds(hp * ETP, ETP)], [e2])
                    sa, sb = plsc.unpack(plsc.bitcast(gs, jnp.bfloat16),
                                         format=plsc.PackFormat.INTERLEAVED)
                    a0, b0 = plsc.unpack(plsc.bitcast(g0, jnp.bfloat16),
                                         format=plsc.PackFormat.INTERLEAVED)
                    a1, b1 = plsc.unpack(plsc.bitcast(g1, jnp.bfloat16),
                                         format=plsc.PackFormat.INTERLEAVED)
                    a2, b2 = plsc.unpack(plsc.bitcast(g2, jnp.bfloat16),
                                         format=plsc.PackFormat.INTERLEAVED)
                    strip_v[p, (2 * hp) * ROWS + ri, pl.ds(j0, L)] = (
                        (sa + a0) + (a1 + a2))
                    strip_v[p, (2 * hp + 1) * ROWS + ri, pl.ds(j0, L)] = (
                        (sb + b0) + (b1 + b2))
            return 0

        lax.fori_loop(0, ROWS, row, 0, unroll=False)

        def fire(h, _):
            pltpu.async_copy(strip_v.at[p, pl.ds(h * ROWS, ROWS), :],
                             out_hbm.at[b, h, pl.ds(i0, ROWS), :], sem)
            return 0

        lax.fori_loop(0, H, fire, 0, unroll=False)

        @pl.when(k < NSTRIP - 1)
        def _():
            pltpu.make_async_copy(spidx_hbm.at[b, pl.ds(0, ROWS), :],
                                  spi_v.at[q], sem_idx).wait()
            pltpu.make_async_copy(
                etidx_hbm.at[b, pl.ds(0, ROWS * N * NEF)],
                eti_v.at[q], sem_idx).wait()
        return 0

    lax.fori_loop(0, NSTRIP, task, 0, unroll=False)

    def tail(k, _):
        p = lax.rem(k, 2)

        def dr(h, _):
            pltpu.make_async_copy(strip_v.at[p, pl.ds(h * ROWS, ROWS), :],
                                  out_hbm.at[b, h, pl.ds(0, ROWS), :],
                                  sem).wait()
            return 0
        lax.fori_loop(0, H, dr, 0, unroll=False)
        return 0

    lax.fori_loop(NSTRIP - 2, NSTRIP, tail, 0, unroll=False)


def _tc_assemble_body(bias_ref, t_ref, tsum_ref, o_ref):
    rr = lax.broadcasted_iota(jnp.int32, (NP1, NP1), 0)
    cc = lax.broadcasted_iota(jnp.int32, (NP1, NP1), 1)
    mask = (rr == 0) | (cc == 0)
    hg = pl.program_id(1)
    for u in range(HB):
        t_u = t_ref[hg * HB + u]
        base = bias_ref[0] + jnp.where(mask, t_u, 0.0)
        o_ref[0, u] = base + jnp.pad(tsum_ref[0, u], ((1, 0), (1, 0)))


_tc_assemble = pl.pallas_call(
    _tc_assemble_body,
    grid=(B, H // HB),
    in_specs=[
        pl.BlockSpec((1, NP1, NP1), lambda b, hg: (b, 0, 0)),
        pl.BlockSpec(memory_space=pltpu.SMEM),
        pl.BlockSpec((1, HB, N, N), lambda b, hg: (b, hg, 0, 0)),
    ],
    out_specs=pl.BlockSpec((1, HB, NP1, NP1), lambda b, hg: (b, hg, 0, 0)),
    out_shape=jax.ShapeDtypeStruct((B, H, NP1, NP1), jnp.float32),
)


def _pack_pairs(T):
    tb = lax.bitcast_convert_type(T.astype(jnp.bfloat16),
                                  jnp.uint16).astype(jnp.uint32)
    lo = tb[0::2, :]
    hi = tb[1::2, :]
    return (lo | (hi << 16)).astype(jnp.int32).reshape(-1)


def kernel(attn_bias, spatial_pos, attn_edge_type, spatial_pos_table,
           edge_table, virtual_dist):
    spT = _pack_pairs(spatial_pos_table.T)
    etT = _pack_pairs(jnp.pad((edge_table * (1.0 / 3.0)).T,
                              ((0, 0), (0, ETP - NET))))
    spi = spatial_pos.astype(jnp.int32)
    eti = attn_edge_type.astype(jnp.int32).reshape(B, N * N * NEF)
    tsum = _sc_tsum_kernel(spi, eti, spT, etT)

    t = virtual_dist.reshape(H).astype(jnp.float32)
    return _tc_assemble(attn_bias, t, tsum)


# speedup vs baseline: 41.2464x; 1.0944x over previous
import functools

import jax
import jax.numpy as jnp
from jax import lax
from jax.experimental import pallas as pl
from jax.experimental.pallas import tpu as pltpu
from jax.experimental.pallas import tpu_sc as plsc

B, N, H = 32, 128, 32
NP1 = N + 1
NSP = 512
NET = 1537
ETP = 1544
NEF = 3
ROWS = 8
NSTRIP = N // ROWS
NC, NS, L = 2, 16, 16
HB = 16

_mesh = plsc.VectorSubcoreMesh(core_axis_name="c", subcore_axis_name="s")


@functools.partial(
    pl.kernel,
    mesh=_mesh,
    compiler_params=pltpu.CompilerParams(use_tc_tiling_on_sc=False,
                                         needs_layout_passes=False),
    out_type=jax.ShapeDtypeStruct((B, H, N, N), jnp.float32),
    scratch_types=[
        pltpu.VMEM((H // 2 * NSP,), jnp.int32),
        pltpu.VMEM((H // 2 * ETP,), jnp.int32),
        pltpu.VMEM((2, ROWS, N), jnp.int32),
        pltpu.VMEM((2, ROWS * N * NEF,), jnp.int32),
        pltpu.VMEM((2, H * ROWS, N), jnp.float32),
        pltpu.SemaphoreType.DMA,
        pltpu.SemaphoreType.DMA,
    ],
)
def _sc_tsum_kernel(spidx_hbm, etidx_hbm, spT_hbm, etT_hbm,
                    out_hbm, spT_v, etT_v, spi_v, eti_v, strip_v,
                    sem, sem_idx):
    w = lax.axis_index("s") * NC + lax.axis_index("c")
    b = w

    pltpu.sync_copy(spT_hbm, spT_v)
    pltpu.sync_copy(etT_hbm, etT_v)

    iota = lax.iota(jnp.int32, L)
    iota3 = iota * 3

    pltpu.sync_copy(spidx_hbm.at[b, pl.ds(0, ROWS), :], spi_v.at[0])
    pltpu.sync_copy(etidx_hbm.at[b, pl.ds(0, ROWS * N * NEF)], eti_v.at[0])

    def task(k, _):
        i0 = ROWS * k
        p = lax.rem(k, 2)
        q = lax.rem(k + 1, 2)

        @pl.when(k < NSTRIP - 1)
        def _():
            pltpu.async_copy(spidx_hbm.at[b, pl.ds(i0 + ROWS, ROWS), :],
                             spi_v.at[q], sem_idx)
            pltpu.async_copy(
                etidx_hbm.at[b, pl.ds((i0 + ROWS) * N * NEF, ROWS * N * NEF)],
                eti_v.at[q], sem_idx)

        @pl.when(k >= 2)
        def _():
            def dr(h, _):
                pltpu.make_async_copy(
                    strip_v.at[p, pl.ds(h * ROWS, ROWS), :],
                    out_hbm.at[b, h, pl.ds(0, ROWS), :], sem).wait()
                return 0
            lax.fori_loop(0, H, dr, 0, unroll=False)

        def row(ri, _):
            for c in range(8):
                j0 = 16 * c
                spi = spi_v[p, ri, pl.ds(j0, L)]
                ebase = iota3 + (ri * (N * NEF) + j0 * NEF)
                e0 = plsc.load_gather(eti_v.at[p], [ebase])
                e1 = plsc.load_gather(eti_v.at[p], [ebase + 1])
                e2 = plsc.load_gather(eti_v.at[p], [ebase + 2])

                @plsc.parallel_loop(0, H // 2, unroll=4)
                def _(hp):
                    gs = plsc.load_gather(
                        spT_v.at[pl.ds(hp * NSP, NSP)], [spi])
                    g0 = plsc.load_gather(
                        etT_v.at[pl.ds(hp * ETP, ETP)], [e0])
                    g1 = plsc.load_gather(
                        etT_v.at[pl.ds(hp * ETP, ETP)], [e1])
                    g2 = plsc.load_gather(
                        etT_v.at[pl.ds(hp * ETP, ETP)], [e2])
                    sa, sb = plsc.unpack(plsc.bitcast(gs, jnp.bfloat16),
                                         format=plsc.PackFormat.INTERLEAVED)
                    a0, b0 = plsc.unpack(plsc.bitcast(g0, jnp.bfloat16),
                                         format=plsc.PackFormat.INTERLEAVED)
                    a1, b1 = plsc.unpack(plsc.bitcast(g1, jnp.bfloat16),
                                         format=plsc.PackFormat.INTERLEAVED)
                    a2, b2 = plsc.unpack(plsc.bitcast(g2, jnp.bfloat16),
                                         format=plsc.PackFormat.INTERLEAVED)
                    strip_v[p, (2 * hp) * ROWS + ri, pl.ds(j0, L)] = (
                        (sa + a0) + (a1 + a2))
                    strip_v[p, (2 * hp + 1) * ROWS + ri, pl.ds(j0, L)] = (
                        (sb + b0) + (b1 + b2))
            return 0

        lax.fori_loop(0, ROWS, row, 0, unroll=False)

        def fire(h, _):
            pltpu.async_copy(strip_v.at[p, pl.ds(h * ROWS, ROWS), :],
                             out_hbm.at[b, h, pl.ds(i0, ROWS), :], sem)
            return 0

        lax.fori_loop(0, H, fire, 0, unroll=False)

        @pl.when(k < NSTRIP - 1)
        def _():
            pltpu.make_async_copy(spidx_hbm.at[b, pl.ds(0, ROWS), :],
                                  spi_v.at[q], sem_idx).wait()
            pltpu.make_async_copy(
                etidx_hbm.at[b, pl.ds(0, ROWS * N * NEF)],
                eti_v.at[q], sem_idx).wait()
        return 0

    lax.fori_loop(0, NSTRIP, task, 0, unroll=False)

    def tail(k, _):
        p = lax.rem(k, 2)

        def dr(h, _):
            pltpu.make_async_copy(strip_v.at[p, pl.ds(h * ROWS, ROWS), :],
                                  out_hbm.at[b, h, pl.ds(0, ROWS), :],
                                  sem).wait()
            return 0
        lax.fori_loop(0, H, dr, 0, unroll=False)
        return 0

    lax.fori_loop(NSTRIP - 2, NSTRIP, tail, 0, unroll=False)


def _tc_assemble_body(bias_ref, t_ref, tsum_ref, o_ref):
    rr = lax.broadcasted_iota(jnp.int32, (NP1, NP1), 0)
    cc = lax.broadcasted_iota(jnp.int32, (NP1, NP1), 1)
    mask = (rr == 0) | (cc == 0)
    hg = pl.program_id(1)
    for u in range(HB):
        t_u = t_ref[hg * HB + u]
        base = bias_ref[0] + jnp.where(mask, t_u, 0.0)
        o_ref[0, u] = base + jnp.pad(tsum_ref[0, u], ((1, 0), (1, 0)))


_tc_assemble = pl.pallas_call(
    _tc_assemble_body,
    grid=(B, H // HB),
    in_specs=[
        pl.BlockSpec((1, NP1, NP1), lambda b, hg: (b, 0, 0)),
        pl.BlockSpec(memory_space=pltpu.SMEM),
        pl.BlockSpec((1, HB, N, N), lambda b, hg: (b, hg, 0, 0)),
    ],
    out_specs=pl.BlockSpec((1, HB, NP1, NP1), lambda b, hg: (b, hg, 0, 0)),
    out_shape=jax.ShapeDtypeStruct((B, H, NP1, NP1), jnp.float32),
)


def _pack_pairs(T):
    tb = lax.bitcast_convert_type(T.astype(jnp.bfloat16),
                                  jnp.uint16).astype(jnp.uint32)
    lo = tb[0::2, :]
    hi = tb[1::2, :]
    return (lo | (hi << 16)).astype(jnp.int32).reshape(-1)


def kernel(attn_bias, spatial_pos, attn_edge_type, spatial_pos_table,
           edge_table, virtual_dist):
    spT = _pack_pairs(spatial_pos_table.T)
    etT = _pack_pairs(jnp.pad((edge_table * (1.0 / 3.0)).T,
                              ((0, 0), (0, ETP - NET))))
    spi = spatial_pos.astype(jnp.int32)
    eti = attn_edge_type.astype(jnp.int32).reshape(B, N * N * NEF)
    tsum = _sc_tsum_kernel(spi, eti, spT, etT)

    t = virtual_dist.reshape(H).astype(jnp.float32)
    return _tc_assemble(attn_bias, t, tsum)


# speedup vs baseline: 43.2234x; 1.0479x over previous
import functools

import jax
import jax.numpy as jnp
from jax import lax
from jax.experimental import pallas as pl
from jax.experimental.pallas import tpu as pltpu
from jax.experimental.pallas import tpu_sc as plsc

B, N, H = 32, 128, 32
NP1 = N + 1
NSP = 512
NET = 1537
ETP = 1544
NEF = 3
ROWS = 8
NSTRIP = N // ROWS
NC, NS, L = 2, 16, 16
HB = 32

_mesh = plsc.VectorSubcoreMesh(core_axis_name="c", subcore_axis_name="s")


@functools.partial(
    pl.kernel,
    mesh=_mesh,
    compiler_params=pltpu.CompilerParams(use_tc_tiling_on_sc=False,
                                         needs_layout_passes=False),
    out_type=jax.ShapeDtypeStruct((B, H, N, N), jnp.float32),
    scratch_types=[
        pltpu.VMEM((H // 2 * NSP,), jnp.int32),
        pltpu.VMEM((H // 2 * ETP,), jnp.int32),
        pltpu.VMEM((2, ROWS, N), jnp.int32),
        pltpu.VMEM((2, ROWS * N * NEF,), jnp.int32),
        pltpu.VMEM((2, H * ROWS, N), jnp.float32),
        pltpu.SemaphoreType.DMA,
        pltpu.SemaphoreType.DMA,
    ],
)
def _sc_tsum_kernel(spidx_hbm, etidx_hbm, spT_hbm, etT_hbm,
                    out_hbm, spT_v, etT_v, spi_v, eti_v, strip_v,
                    sem, sem_idx):
    w = lax.axis_index("s") * NC + lax.axis_index("c")
    b = w

    pltpu.sync_copy(spT_hbm, spT_v)
    pltpu.sync_copy(etT_hbm, etT_v)

    iota = lax.iota(jnp.int32, L)
    iota3 = iota * 3

    pltpu.sync_copy(spidx_hbm.at[b, pl.ds(0, ROWS), :], spi_v.at[0])
    pltpu.sync_copy(etidx_hbm.at[b, pl.ds(0, ROWS * N * NEF)], eti_v.at[0])

    def task(k, _):
        i0 = ROWS * k
        p = lax.rem(k, 2)
        q = lax.rem(k + 1, 2)

        @pl.when(k < NSTRIP - 1)
        def _():
            pltpu.async_copy(spidx_hbm.at[b, pl.ds(i0 + ROWS, ROWS), :],
                             spi_v.at[q], sem_idx)
            pltpu.async_copy(
                etidx_hbm.at[b, pl.ds((i0 + ROWS) * N * NEF, ROWS * N * NEF)],
                eti_v.at[q], sem_idx)

        @pl.when(k >= 2)
        def _():
            def dr(h, _):
                pltpu.make_async_copy(
                    strip_v.at[p, pl.ds(h * ROWS, ROWS), :],
                    out_hbm.at[b, h, pl.ds(0, ROWS), :], sem).wait()
                return 0
            lax.fori_loop(0, H, dr, 0, unroll=False)

        def row(ri, _):
            for c in range(8):
                j0 = 16 * c
                spi = spi_v[p, ri, pl.ds(j0, L)]
                ebase = iota3 + (ri * (N * NEF) + j0 * NEF)
                e0 = plsc.load_gather(eti_v.at[p], [ebase])
                e1 = plsc.load_gather(eti_v.at[p], [ebase + 1])
                e2 = plsc.load_gather(eti_v.at[p], [ebase + 2])

                @plsc.parallel_loop(0, H // 2, unroll=4)
                def _(hp):
                    gs = plsc.load_gather(
                        spT_v.at[pl.ds(hp * NSP, NSP)], [spi])
                    g0 = plsc.load_gather(
                        etT_v.at[pl.ds(hp * ETP, ETP)], [e0])
                    g1 = plsc.load_gather(
                        etT_v.at[pl.ds(hp * ETP, ETP)], [e1])
                    g2 = plsc.load_gather(
                        etT_v.at[pl.ds(hp * ETP, ETP)], [e2])
                    sa, sb = plsc.unpack(plsc.bitcast(gs, jnp.bfloat16),
                                         format=plsc.PackFormat.INTERLEAVED)
                    a0, b0 = plsc.unpack(plsc.bitcast(g0, jnp.bfloat16),
                                         format=plsc.PackFormat.INTERLEAVED)
                    a1, b1 = plsc.unpack(plsc.bitcast(g1, jnp.bfloat16),
                                         format=plsc.PackFormat.INTERLEAVED)
                    a2, b2 = plsc.unpack(plsc.bitcast(g2, jnp.bfloat16),
                                         format=plsc.PackFormat.INTERLEAVED)
                    strip_v[p, (2 * hp) * ROWS + ri, pl.ds(j0, L)] = (
                        (sa + a0) + (a1 + a2))
                    strip_v[p, (2 * hp + 1) * ROWS + ri, pl.ds(j0, L)] = (
                        (sb + b0) + (b1 + b2))
            return 0

        lax.fori_loop(0, ROWS, row, 0, unroll=False)

        def fire(h, _):
            pltpu.async_copy(strip_v.at[p, pl.ds(h * ROWS, ROWS), :],
                             out_hbm.at[b, h, pl.ds(i0, ROWS), :], sem)
            return 0

        lax.fori_loop(0, H, fire, 0, unroll=False)

        @pl.when(k < NSTRIP - 1)
        def _():
            pltpu.make_async_copy(spidx_hbm.at[b, pl.ds(0, ROWS), :],
                                  spi_v.at[q], sem_idx).wait()
            pltpu.make_async_copy(
                etidx_hbm.at[b, pl.ds(0, ROWS * N * NEF)],
                eti_v.at[q], sem_idx).wait()
        return 0

    lax.fori_loop(0, NSTRIP, task, 0, unroll=False)

    def tail(k, _):
        p = lax.rem(k, 2)

        def dr(h, _):
            pltpu.make_async_copy(strip_v.at[p, pl.ds(h * ROWS, ROWS), :],
                                  out_hbm.at[b, h, pl.ds(0, ROWS), :],
                                  sem).wait()
            return 0
        lax.fori_loop(0, H, dr, 0, unroll=False)
        return 0

    lax.fori_loop(NSTRIP - 2, NSTRIP, tail, 0, unroll=False)


def _tc_assemble_body(bias_ref, t_ref, tsum_ref, o_ref):
    rr = lax.broadcasted_iota(jnp.int32, (NP1, NP1), 0)
    cc = lax.broadcasted_iota(jnp.int32, (NP1, NP1), 1)
    mask = (rr == 0) | (cc == 0)
    hg = pl.program_id(1)
    for u in range(HB):
        t_u = t_ref[hg * HB + u]
        base = bias_ref[0] + jnp.where(mask, t_u, 0.0)
        o_ref[0, u] = base + jnp.pad(tsum_ref[0, u], ((1, 0), (1, 0)))


_tc_assemble = pl.pallas_call(
    _tc_assemble_body,
    grid=(B, H // HB),
    in_specs=[
        pl.BlockSpec((1, NP1, NP1), lambda b, hg: (b, 0, 0)),
        pl.BlockSpec(memory_space=pltpu.SMEM),
        pl.BlockSpec((1, HB, N, N), lambda b, hg: (b, hg, 0, 0)),
    ],
    out_specs=pl.BlockSpec((1, HB, NP1, NP1), lambda b, hg: (b, hg, 0, 0)),
    out_shape=jax.ShapeDtypeStruct((B, H, NP1, NP1), jnp.float32),
)


def _pack_pairs(T):
    tb = lax.bitcast_convert_type(T.astype(jnp.bfloat16),
                                  jnp.uint16).astype(jnp.uint32)
    lo = tb[0::2, :]
    hi = tb[1::2, :]
    return (lo | (hi << 16)).astype(jnp.int32).reshape(-1)


def kernel(attn_bias, spatial_pos, attn_edge_type, spatial_pos_table,
           edge_table, virtual_dist):
    spT = _pack_pairs(spatial_pos_table.T)
    etT = _pack_pairs(jnp.pad((edge_table * (1.0 / 3.0)).T,
                              ((0, 0), (0, ETP - NET))))
    spi = spatial_pos.astype(jnp.int32)
    eti = attn_edge_type.astype(jnp.int32).reshape(B, N * N * NEF)
    tsum = _sc_tsum_kernel(spi, eti, spT, etT)

    t = virtual_dist.reshape(H).astype(jnp.float32)
    return _tc_assemble(attn_bias, t, tsum)
